# baseline (speedup 1.0000x reference)
import functools

import jax
import jax.numpy as jnp
from jax import lax
from jax.experimental import pallas as pl
from jax.experimental.pallas import tpu as pltpu
from jax.experimental.pallas import tpu_sc as plsc

N = 489
NP = 512
FEAT = 512
H1 = 256
H2 = 128
E = 31296
EP = 32768
EC = 2048
DRUG = 271
MIC = N - DRUG
GAMMA = 0.5

NSUB = 16
EPS = EP // NSUB
TBL = NP * NP
TSLICE = TBL // NSUB
ZCH = 2048

_OUT_SHAPES = (
    jax.ShapeDtypeStruct((DRUG, MIC), jnp.float32),
    jax.ShapeDtypeStruct((N, 2), jnp.float32),
    jax.ShapeDtypeStruct((N, 2), jnp.float32),
    jax.ShapeDtypeStruct((N, H1 * 2 // 2), jnp.float32),
)


@functools.cache
def _adj_sc():
    mesh = plsc.VectorSubcoreMesh(core_axis_name="c", subcore_axis_name="s")
    return pl.kernel(
        _adj_sc_body,
        out_type=jax.ShapeDtypeStruct((2, TBL), jnp.float32),
        mesh=mesh,
        scratch_types=[
            pltpu.VMEM((EPS,), jnp.int32),
            pltpu.VMEM((EPS,), jnp.int32),
            pltpu.VMEM((EPS // 128, 128), jnp.int32),
            pltpu.VMEM((128,), jnp.float32),
            pltpu.VMEM((ZCH,), jnp.float32),
            pltpu.VMEM_SHARED((TBL,), jnp.float32),
        ],
    )


def _adj_sc_body(ei_hbm, out_hbm, srcb, dstb, idxb, onesb, zerob, table):
    cid = lax.axis_index("c")
    sid = lax.axis_index("s")

    zeros16 = jnp.zeros((16,), jnp.float32)
    ones16 = jnp.ones((16,), jnp.float32)
    for i in range(ZCH // 16):
        zerob[pl.ds(i * 16, 16)] = zeros16
    for i in range(128 // 16):
        onesb[pl.ds(i * 16, 16)] = ones16

    for j in range(TSLICE // ZCH):
        pltpu.sync_copy(zerob, table.at[pl.ds(sid * TSLICE + j * ZCH, ZCH)])

    pltpu.sync_copy(ei_hbm.at[cid, 0, pl.ds(sid * EPS, EPS)], srcb)
    pltpu.sync_copy(ei_hbm.at[cid, 1, pl.ds(sid * EPS, EPS)], dstb)
    for r in range(EPS // 128):
        for i in range(8):
            off = r * 128 + i * 16
            s16 = srcb[pl.ds(off, 16)]
            d16 = dstb[pl.ds(off, 16)]
            idxb[r, pl.ds(i * 16, 16)] = d16 * NP + s16

    plsc.subcore_barrier()

    for r in range(EPS // 128):
        pltpu.sync_copy(onesb, table.at[idxb.at[r]], add=True)

    plsc.subcore_barrier()

    pltpu.sync_copy(table.at[pl.ds(sid * TSLICE, TSLICE)],
                    out_hbm.at[cid, pl.ds(sid * TSLICE, TSLICE)])


def _dot(a, b):
    return lax.dot_general(a, b, (((1,), (0,)), ((), ())),
                           preferred_element_type=jnp.float32)


def _dot_t(a, b):
    return lax.dot_general(a, b, (((1,), (1,)), ((), ())),
                           preferred_element_type=jnp.float32)


def _gip(y, m):
    mn = jnp.min(y, axis=1, keepdims=True)
    mx = jnp.max(y, axis=1, keepdims=True)
    yn = (y - mn) / (mx - mn)
    k = _dot_t(yn, yn)
    r = lax.broadcasted_iota(jnp.int32, (m, m), 0)
    c = lax.broadcasted_iota(jnp.int32, (m, m), 1)
    eye = (r == c).astype(jnp.float32)
    dcol = jnp.sum(k * eye, axis=1, keepdims=True)
    md = jnp.sum(dcol) / m
    k = k / md
    dcol = dcol / md
    drow = jnp.sum(k * eye, axis=0, keepdims=True)
    dist = dcol + drow - 2.0 * k
    kk = jnp.abs(jnp.exp(dist * (-GAMMA)))
    mp = jnp.min(jnp.where(kk > 0.0, kk, jnp.inf))
    min_v = jnp.where(mp == jnp.inf, 0.0, mp)
    kk = jnp.where(kk == 0.0, min_v, kk)
    dg_c = jnp.sqrt(jnp.sum(kk * eye, axis=1, keepdims=True))
    dg_r = jnp.sqrt(jnp.sum(kk * eye, axis=0, keepdims=True))
    return kk / (dg_c * dg_r)


def _mngnn_kernel(xo_ref, xa_ref, adjo_ref, adjs_ref,
                  Wo1_ref, Ws1_ref, Wo2_ref, Ws2_ref, dW_ref,
                  b1o_ref, b1s_ref, b2o_ref, b2s_ref, db_ref,
                  a1_ref, a2_ref,
                  out_ref, ros_ref, rosa_ref, x2_ref):
    r = lax.broadcasted_iota(jnp.int32, (NP, NP), 0)
    c = lax.broadcasted_iota(jnp.int32, (NP, NP), 1)
    eye = (r == c).astype(jnp.float32)

    def make_A(adj):
        deg = jnp.sum(adj, axis=1, keepdims=True) + 1.0
        dinv = 1.0 / jnp.sqrt(deg)
        dinv_r = jnp.sum(dinv * eye, axis=0, keepdims=True)
        return adj * dinv * dinv_r + eye * (dinv * dinv)

    A_o = make_A(adjo_ref[:])
    A_s = make_A(adjs_ref[:])

    Wo1 = Wo1_ref[:]
    Ws1 = Ws1_ref[:]
    Wo2a = Wo2_ref[0:H1, :]
    Wo2b = Wo2_ref[H1:2 * H1, :]
    Ws2a = Ws2_ref[0:H1, :]
    Ws2b = Ws2_ref[H1:2 * H1, :]
    b1o = b1o_ref[:]
    b1s = b1s_ref[:]
    b2o = b2o_ref[:]
    b2s = b2s_ref[:]

    def gnn(x):
        x1o = jnp.maximum(_dot(A_o, _dot(x, Wo1)) + b1o, 0.0)
        x1s = jnp.maximum(_dot(A_s, _dot(x, Ws1)) + b1s, 0.0)
        x2o = _dot(A_o, _dot(x1o, Wo2a) + _dot(x1s, Wo2b)) + b2o
        x2s = _dot(A_s, _dot(x1o, Ws2a) + _dot(x1s, Ws2b)) + b2s
        return jnp.concatenate([x2o, x2s], axis=1)

    x2 = gnn(xo_ref[:])
    x2a = gnn(xa_ref[:])

    rowmask = (lax.broadcasted_iota(jnp.int32, (NP, 1), 0) < N).astype(jnp.float32)
    h = jax.nn.sigmoid(jnp.sum(x2 * rowmask, axis=0, keepdims=True) / N)
    ha = jax.nn.sigmoid(jnp.sum(x2a * rowmask, axis=0, keepdims=True) / N)

    dW = dW_ref[:]
    db = db_ref[:]
    v = _dot_t(h, dW)
    va = _dot_t(ha, dW)
    sc1 = _dot_t(x2, v)
    sc2 = _dot_t(x2a, v)
    sc1a = _dot_t(x2a, va)
    sc2a = _dot_t(x2, va)
    ros = jnp.concatenate([sc1, sc2], axis=1) + db
    rosa = jnp.concatenate([sc1a, sc2a], axis=1) + db
    ros_ref[:] = ros[0:N, :]
    rosa_ref[:] = rosa[0:N, :]

    drug_k = _gip(x2[0:DRUG, :], DRUG)
    mic_k = _gip(x2[DRUG:N, :], MIC)
    out1 = _dot(drug_k, a1_ref[:])
    out2t = lax.dot_general(a2_ref[:], mic_k, (((0,), (0,)), ((), ())),
                            preferred_element_type=jnp.float32)
    out_ref[:] = (out1 + out2t) * 0.5
    x2_ref[:] = x2[0:N, :]


def _prep(x_o, x_a, adj_o, adj_s, W_o1, b_o1, W_s1, b_s1,
          W_o2, b_o2, W_s2, b_s2, disc_W, disc_b, alpha1, alpha2):
    xo_p = jnp.pad(x_o, ((0, NP - N), (0, 0)))
    xa_p = jnp.pad(x_a, ((0, NP - N), (0, 0)))
    return (xo_p, xa_p, adj_o, adj_s,
            W_o1, W_s1, W_o2, W_s2, disc_W,
            b_o1.reshape(1, H1), b_s1.reshape(1, H1),
            b_o2.reshape(1, H2), b_s2.reshape(1, H2),
            disc_b.reshape(1, 1), alpha1, alpha2)


@jax.jit
def _fwd(x_o, x_a, edge_index_o, edge_index_s, W_o1, b_o1, W_s1, b_s1,
         W_o2, b_o2, W_s2, b_s2, disc_W, disc_b, alpha1, alpha2):
    eio_p = jnp.pad(edge_index_o, ((0, 0), (0, EP - E)), constant_values=NP - 1)
    eis_p = jnp.pad(edge_index_s, ((0, 0), (0, EP - E)), constant_values=NP - 1)
    ei_all = jnp.stack([eio_p, eis_p])
    adj = _adj_sc()(ei_all)
    adj_o = adj[0].reshape(NP, NP)
    adj_s = adj[1].reshape(NP, NP)
    return pl.pallas_call(_mngnn_kernel, out_shape=_OUT_SHAPES)(*_prep(
        x_o, x_a, adj_o, adj_s, W_o1, b_o1, W_s1, b_s1,
        W_o2, b_o2, W_s2, b_s2, disc_W, disc_b, alpha1, alpha2))


def kernel(x_o, x_a, edge_index_o, edge_index_s, W_o1, b_o1, W_s1, b_s1,
           W_o2, b_o2, W_s2, b_s2, disc_W, disc_b, alpha1, alpha2):
    out, ros, rosa, x2 = _fwd(
        x_o, x_a, edge_index_o, edge_index_s, W_o1, b_o1, W_s1, b_s1,
        W_o2, b_o2, W_s2, b_s2, disc_W, disc_b, alpha1, alpha2)
    return (out, ros, rosa, x2)


# speedup vs baseline: 1.3994x; 1.3994x over previous
import functools

import jax
import jax.numpy as jnp
from jax import lax
from jax.experimental import pallas as pl
from jax.experimental.pallas import tpu as pltpu
from jax.experimental.pallas import tpu_sc as plsc

N = 489
NP = 512
FEAT = 512
H1 = 256
H2 = 128
E = 31296
DRUG = 271
MIC = N - DRUG
GAMMA = 0.5

NSUB = 16
EPS = 1960
EPS_LAST = E - 15 * EPS
GROUPS = 123
TBL = NP * NP
TSLICE = TBL // NSUB
ZCH = 2048
SENT = TBL - 1


@functools.cache
def _adj_sc():
    mesh = plsc.VectorSubcoreMesh(core_axis_name="c", subcore_axis_name="s")
    return pl.kernel(
        _adj_sc_body,
        out_type=(jax.ShapeDtypeStruct((TBL,), jnp.float32),
                  jax.ShapeDtypeStruct((TBL,), jnp.float32)),
        mesh=mesh,
        scratch_types=[
            pltpu.VMEM((2048,), jnp.int32),
            pltpu.VMEM((2048,), jnp.int32),
            pltpu.VMEM((NSUB, 128), jnp.int32),
            pltpu.VMEM((128,), jnp.float32),
            pltpu.VMEM((ZCH,), jnp.float32),
            pltpu.VMEM_SHARED((TBL,), jnp.float32),
            pltpu.SemaphoreType.DMA,
            pltpu.SemaphoreType.DMA,
            pltpu.SemaphoreType.DMA,
        ],
    )


def _adj_sc_body(eio_hbm, eis_hbm, out0_hbm, out1_hbm,
                 srcb, dstb, idxb, onesb, zerob, table,
                 sem_z, sem_e, sem_s):
    cid = lax.axis_index("c")
    sid = lax.axis_index("s")

    zeros16 = jnp.zeros((16,), jnp.float32)
    ones16 = jnp.ones((16,), jnp.float32)
    for i in range(ZCH // 16):
        zerob[pl.ds(i * 16, 16)] = zeros16
    for i in range(128 // 16):
        onesb[pl.ds(i * 16, 16)] = ones16

    for j in range(TSLICE // ZCH):
        pltpu.async_copy(zerob, table.at[pl.ds(sid * TSLICE + j * ZCH, ZCH)],
                         sem_z)

    last = sid == NSUB - 1
    base = sid * EPS
    cnt = jnp.where(last, EPS_LAST, EPS)

    @pl.when(cid == 0)
    def _():
        pltpu.async_copy(eio_hbm.at[pl.ds(base, EPS_LAST)],
                         srcb.at[pl.ds(0, EPS_LAST)], sem_e)
        pltpu.async_copy(eio_hbm.at[pl.ds(E + base, EPS_LAST)],
                         dstb.at[pl.ds(0, EPS_LAST)], sem_e)

    @pl.when(cid == 1)
    def _():
        pltpu.async_copy(eis_hbm.at[pl.ds(base, EPS_LAST)],
                         srcb.at[pl.ds(0, EPS_LAST)], sem_e)
        pltpu.async_copy(eis_hbm.at[pl.ds(E + base, EPS_LAST)],
                         dstb.at[pl.ds(0, EPS_LAST)], sem_e)

    @pl.when(jnp.logical_and(cid == 0, jnp.logical_not(last)))
    def _():
        pltpu.sync_copy(eio_hbm.at[pl.ds(base + EPS_LAST, EPS - EPS_LAST)],
                        srcb.at[pl.ds(EPS_LAST, EPS - EPS_LAST)])
        pltpu.sync_copy(eio_hbm.at[pl.ds(E + base + EPS_LAST, EPS - EPS_LAST)],
                        dstb.at[pl.ds(EPS_LAST, EPS - EPS_LAST)])

    @pl.when(jnp.logical_and(cid == 1, jnp.logical_not(last)))
    def _():
        pltpu.sync_copy(eis_hbm.at[pl.ds(base + EPS_LAST, EPS - EPS_LAST)],
                        srcb.at[pl.ds(EPS_LAST, EPS - EPS_LAST)])
        pltpu.sync_copy(eis_hbm.at[pl.ds(E + base + EPS_LAST, EPS - EPS_LAST)],
                        dstb.at[pl.ds(EPS_LAST, EPS - EPS_LAST)])

    pltpu.make_async_copy(eio_hbm.at[pl.ds(0, EPS_LAST)],
                          srcb.at[pl.ds(0, EPS_LAST)], sem_e).wait()
    pltpu.make_async_copy(eio_hbm.at[pl.ds(0, EPS_LAST)],
                          dstb.at[pl.ds(0, EPS_LAST)], sem_e).wait()

    lane = lax.iota(jnp.int32, 16)
    sent16 = jnp.full((16,), SENT, jnp.int32)
    for g in range(GROUPS):
        s16 = srcb[pl.ds(g * 16, 16)]
        d16 = dstb[pl.ds(g * 16, 16)]
        valid = (g * 16 + lane) < cnt
        idx = jnp.where(valid, d16 * NP + s16, sent16)
        idxb[g // 8, pl.ds((g % 8) * 16, 16)] = idx
    for g in range(GROUPS, 128):
        idxb[g // 8, pl.ds((g % 8) * 16, 16)] = sent16

    for j in range(TSLICE // ZCH):
        pltpu.make_async_copy(zerob, table.at[pl.ds(j * ZCH, ZCH)],
                              sem_z).wait()
    plsc.subcore_barrier()

    for r in range(NSUB):
        pltpu.async_copy(onesb, table.at[idxb.at[r]], sem_s, add=True)
    for r in range(NSUB):
        pltpu.make_async_copy(onesb, table.at[idxb.at[0]], sem_s).wait()

    plsc.subcore_barrier()

    @pl.when(cid == 0)
    def _():
        pltpu.sync_copy(table.at[pl.ds(sid * TSLICE, TSLICE)],
                        out0_hbm.at[pl.ds(sid * TSLICE, TSLICE)])

    @pl.when(cid == 1)
    def _():
        pltpu.sync_copy(table.at[pl.ds(sid * TSLICE, TSLICE)],
                        out1_hbm.at[pl.ds(sid * TSLICE, TSLICE)])


_PRE_SHAPES = tuple(jax.ShapeDtypeStruct((N, H1), jnp.float32)
                    for _ in range(4))

_OUT_SHAPES = (
    jax.ShapeDtypeStruct((DRUG, MIC), jnp.float32),
    jax.ShapeDtypeStruct((N, 2), jnp.float32),
    jax.ShapeDtypeStruct((N, 2), jnp.float32),
    jax.ShapeDtypeStruct((N, H1), jnp.float32),
)


def _dot(a, b):
    return lax.dot_general(a, b, (((1,), (0,)), ((), ())),
                           preferred_element_type=jnp.float32)


def _dot_t(a, b):
    return lax.dot_general(a, b, (((1,), (1,)), ((), ())),
                           preferred_element_type=jnp.float32)


def _pre_kernel(xo_ref, xa_ref, Wo1_ref, Ws1_ref,
                xwoo_ref, xwos_ref, xwao_ref, xwas_ref):
    xo = xo_ref[:]
    xa = xa_ref[:]
    Wo1 = Wo1_ref[:]
    Ws1 = Ws1_ref[:]
    xwoo_ref[:] = _dot(xo, Wo1)
    xwos_ref[:] = _dot(xo, Ws1)
    xwao_ref[:] = _dot(xa, Wo1)
    xwas_ref[:] = _dot(xa, Ws1)


def _gip(y, m):
    mn = jnp.min(y, axis=1, keepdims=True)
    mx = jnp.max(y, axis=1, keepdims=True)
    yn = (y - mn) / (mx - mn)
    k = _dot_t(yn, yn)
    r = lax.broadcasted_iota(jnp.int32, (m, m), 0)
    c = lax.broadcasted_iota(jnp.int32, (m, m), 1)
    eye = (r == c).astype(jnp.float32)
    dcol = jnp.sum(k * eye, axis=1, keepdims=True)
    md = jnp.sum(dcol) / m
    k = k / md
    dcol = dcol / md
    drow = jnp.sum(k * eye, axis=0, keepdims=True)
    dist = dcol + drow - 2.0 * k
    kk = jnp.abs(jnp.exp(dist * (-GAMMA)))
    mp = jnp.min(jnp.where(kk > 0.0, kk, jnp.inf))
    min_v = jnp.where(mp == jnp.inf, 0.0, mp)
    kk = jnp.where(kk == 0.0, min_v, kk)
    dg_c = jnp.sqrt(jnp.sum(kk * eye, axis=1, keepdims=True))
    dg_r = jnp.sqrt(jnp.sum(kk * eye, axis=0, keepdims=True))
    return kk / (dg_c * dg_r)


def _mngnn_kernel(adjo_ref, adjs_ref,
                  xwoo_ref, xwos_ref, xwao_ref, xwas_ref,
                  Wo2_ref, Ws2_ref, dW_ref,
                  b1o_ref, b1s_ref, b2o_ref, b2s_ref, db_ref,
                  a1_ref, a2r_ref,
                  out_ref, ros_ref, rosa_ref, x2_ref):
    r = lax.broadcasted_iota(jnp.int32, (N, N), 0)
    c = lax.broadcasted_iota(jnp.int32, (N, N), 1)
    eye = (r == c).astype(jnp.float32)

    def make_A(adj_ref):
        adj = adj_ref[0:N, 0:N]
        deg = jnp.sum(adj, axis=1, keepdims=True) + 1.0
        dinv = 1.0 / jnp.sqrt(deg)
        dinv_r = jnp.sum(dinv * eye, axis=0, keepdims=True)
        return adj * dinv * dinv_r + eye * (dinv * dinv)

    A_o = make_A(adjo_ref)
    A_s = make_A(adjs_ref)

    Wo2a = Wo2_ref[0:H1, :]
    Wo2b = Wo2_ref[H1:2 * H1, :]
    Ws2a = Ws2_ref[0:H1, :]
    Ws2b = Ws2_ref[H1:2 * H1, :]
    b1o = b1o_ref[:]
    b1s = b1s_ref[:]
    b2o = b2o_ref[:]
    b2s = b2s_ref[:]

    def gnn(xwo, xws):
        x1o = jnp.maximum(_dot(A_o, xwo) + b1o, 0.0)
        x1s = jnp.maximum(_dot(A_s, xws) + b1s, 0.0)
        x2o = _dot(A_o, _dot(x1o, Wo2a) + _dot(x1s, Wo2b)) + b2o
        x2s = _dot(A_s, _dot(x1o, Ws2a) + _dot(x1s, Ws2b)) + b2s
        return jnp.concatenate([x2o, x2s], axis=1)

    x2 = gnn(xwoo_ref[:], xwos_ref[:])
    x2a = gnn(xwao_ref[:], xwas_ref[:])

    h = jax.nn.sigmoid(jnp.sum(x2, axis=0, keepdims=True) / N)
    ha = jax.nn.sigmoid(jnp.sum(x2a, axis=0, keepdims=True) / N)

    dW = dW_ref[:]
    db = db_ref[:]
    v = _dot_t(h, dW)
    va = _dot_t(ha, dW)
    sc1 = _dot_t(x2, v)
    sc2 = _dot_t(x2a, v)
    sc1a = _dot_t(x2a, va)
    sc2a = _dot_t(x2, va)
    ros_ref[:] = jnp.concatenate([sc1, sc2], axis=1) + db
    rosa_ref[:] = jnp.concatenate([sc1a, sc2a], axis=1) + db

    drug_k = _gip(x2[0:DRUG, :], DRUG)
    mic_k = _gip(x2[DRUG:N, :], MIC)
    out1 = _dot(drug_k, a1_ref[:])
    out2t = _dot(a2r_ref[:], mic_k)
    out_ref[:] = (out1 + out2t) * 0.5
    x2_ref[:] = x2


@jax.jit
def _fwd(x_o, x_a, edge_index_o, edge_index_s, W_o1, b_o1, W_s1, b_s1,
         W_o2, b_o2, W_s2, b_s2, disc_W, disc_b, alpha1, alpha2):
    adj0, adj1 = _adj_sc()(edge_index_o.reshape(-1),
                           edge_index_s.reshape(-1))
    xw = pl.pallas_call(_pre_kernel, out_shape=_PRE_SHAPES)(
        x_o, x_a, W_o1, W_s1)
    return pl.pallas_call(_mngnn_kernel, out_shape=_OUT_SHAPES)(
        adj0.reshape(NP, NP), adj1.reshape(NP, NP), *xw,
        W_o2, W_s2, disc_W,
        b_o1.reshape(1, H1), b_s1.reshape(1, H1),
        b_o2.reshape(1, H2), b_s2.reshape(1, H2),
        disc_b.reshape(1, 1), alpha1, alpha2.T)


def kernel(x_o, x_a, edge_index_o, edge_index_s, W_o1, b_o1, W_s1, b_s1,
           W_o2, b_o2, W_s2, b_s2, disc_W, disc_b, alpha1, alpha2):
    out, ros, rosa, x2 = _fwd(
        x_o, x_a, edge_index_o, edge_index_s, W_o1, b_o1, W_s1, b_s1,
        W_o2, b_o2, W_s2, b_s2, disc_W, disc_b, alpha1, alpha2)
    return (out, ros, rosa, x2)


# speedup vs baseline: 1.4975x; 1.0701x over previous
import functools

import jax
import jax.numpy as jnp
from jax import lax
from jax.experimental import pallas as pl
from jax.experimental.pallas import tpu as pltpu
from jax.experimental.pallas import tpu_sc as plsc

N = 489
NP = 512
FEAT = 512
H1 = 256
H2 = 128
E = 31296
DRUG = 271
MIC = N - DRUG
GAMMA = 0.5

NSUB = 16
EPAD = 32768
EPS = EPAD // NSUB
IRW = 16
TBL = NP * NP
TSLICE = TBL // NSUB
ZCH = 2048
SENT = TBL - 1


@functools.cache
def _adj_sc():
    mesh = plsc.VectorSubcoreMesh(core_axis_name="c", subcore_axis_name="s")
    return pl.kernel(
        _adj_sc_body,
        out_type=(jax.ShapeDtypeStruct((TBL,), jnp.float32),
                  jax.ShapeDtypeStruct((TBL,), jnp.float32)),
        mesh=mesh,
        scratch_types=[
            pltpu.VMEM((EPS,), jnp.int32),
            pltpu.VMEM((IRW, 128), jnp.int32),
            pltpu.VMEM((128,), jnp.float32),
            pltpu.VMEM((ZCH,), jnp.float32),
            pltpu.VMEM_SHARED((TBL,), jnp.float32),
            pltpu.SemaphoreType.DMA,
            pltpu.SemaphoreType.DMA,
            pltpu.SemaphoreType.DMA,
        ],
    )


def _adj_sc_body(idx_hbm, out0_hbm, out1_hbm,
                 idx1d, idxb, onesb, zerob, table,
                 sem_z, sem_e, sem_s):
    cid = lax.axis_index("c")
    sid = lax.axis_index("s")

    zeros16 = jnp.zeros((16,), jnp.float32)
    ones16 = jnp.ones((16,), jnp.float32)
    for i in range(ZCH // 16):
        zerob[pl.ds(i * 16, 16)] = zeros16
    for i in range(128 // 16):
        onesb[pl.ds(i * 16, 16)] = ones16

    for j in range(TSLICE // ZCH):
        pltpu.async_copy(zerob, table.at[pl.ds(sid * TSLICE + j * ZCH, ZCH)],
                         sem_z)

    base = cid * EPAD + sid * EPS
    pltpu.async_copy(idx_hbm.at[pl.ds(base, EPS)],
                     idx1d.at[pl.ds(0, EPS)], sem_e)
    pltpu.make_async_copy(idx_hbm.at[pl.ds(0, EPS)],
                          idx1d.at[pl.ds(0, EPS)], sem_e).wait()

    for g in range(EPS // 16):
        idxb[g // 8, pl.ds((g % 8) * 16, 16)] = idx1d[pl.ds(g * 16, 16)]

    for j in range(TSLICE // ZCH):
        pltpu.make_async_copy(zerob, table.at[pl.ds(j * ZCH, ZCH)],
                              sem_z).wait()
    plsc.subcore_barrier()

    for r in range(IRW):
        pltpu.async_copy(onesb, table.at[idxb.at[r]], sem_s, add=True)
    for r in range(IRW):
        pltpu.make_async_copy(onesb, table.at[idxb.at[0]], sem_s).wait()

    plsc.subcore_barrier()

    @pl.when(cid == 0)
    def _():
        pltpu.sync_copy(table.at[pl.ds(sid * TSLICE, TSLICE)],
                        out0_hbm.at[pl.ds(sid * TSLICE, TSLICE)])

    @pl.when(cid == 1)
    def _():
        pltpu.sync_copy(table.at[pl.ds(sid * TSLICE, TSLICE)],
                        out1_hbm.at[pl.ds(sid * TSLICE, TSLICE)])


_PRE_SHAPES = tuple(jax.ShapeDtypeStruct((N, H1), jnp.float32)
                    for _ in range(4))

_OUT_SHAPES = (
    jax.ShapeDtypeStruct((DRUG, MIC), jnp.float32),
    jax.ShapeDtypeStruct((N, 2), jnp.float32),
    jax.ShapeDtypeStruct((N, 2), jnp.float32),
    jax.ShapeDtypeStruct((N, H1), jnp.float32),
)


def _dot(a, b):
    return lax.dot_general(a, b, (((1,), (0,)), ((), ())),
                           preferred_element_type=jnp.float32)


def _dot_t(a, b):
    return lax.dot_general(a, b, (((1,), (1,)), ((), ())),
                           preferred_element_type=jnp.float32)


_IDX_SHAPES = jax.ShapeDtypeStruct((2 * EPAD,), jnp.int32)


def _idx_kernel(eio_ref, eis_ref, io_ref):
    pad = jnp.full((1, EPAD - E), SENT, jnp.int32)
    for half, ref in enumerate((eio_ref, eis_ref)):
        idx = ref[1:2, :] * NP + ref[0:1, :]
        io_ref[pl.ds(half * EPAD, EPAD)] = jnp.concatenate(
            [idx, pad], axis=1).reshape(EPAD)


def _pre_kernel(xo_ref, xa_ref, Wo1_ref, Ws1_ref,
                xwoo_ref, xwos_ref, xwao_ref, xwas_ref):
    xo = xo_ref[:]
    xa = xa_ref[:]
    Wo1 = Wo1_ref[:]
    Ws1 = Ws1_ref[:]
    xwoo_ref[:] = _dot(xo, Wo1)
    xwos_ref[:] = _dot(xo, Ws1)
    xwao_ref[:] = _dot(xa, Wo1)
    xwas_ref[:] = _dot(xa, Ws1)


def _gip(y, m):
    mn = jnp.min(y, axis=1, keepdims=True)
    mx = jnp.max(y, axis=1, keepdims=True)
    yn = (y - mn) / (mx - mn)
    k = _dot_t(yn, yn)
    r = lax.broadcasted_iota(jnp.int32, (m, m), 0)
    c = lax.broadcasted_iota(jnp.int32, (m, m), 1)
    eye = (r == c).astype(jnp.float32)
    dcol = jnp.sum(k * eye, axis=1, keepdims=True)
    md = jnp.sum(dcol) / m
    k = k / md
    dcol = dcol / md
    drow = jnp.sum(k * eye, axis=0, keepdims=True)
    dist = dcol + drow - 2.0 * k
    kk = jnp.abs(jnp.exp(dist * (-GAMMA)))
    mp = jnp.min(jnp.where(kk > 0.0, kk, jnp.inf))
    min_v = jnp.where(mp == jnp.inf, 0.0, mp)
    kk = jnp.where(kk == 0.0, min_v, kk)
    dg_c = jnp.sqrt(jnp.sum(kk * eye, axis=1, keepdims=True))
    dg_r = jnp.sqrt(jnp.sum(kk * eye, axis=0, keepdims=True))
    return kk / (dg_c * dg_r)


def _mngnn_kernel(adjo_ref, adjs_ref,
                  xwoo_ref, xwos_ref, xwao_ref, xwas_ref,
                  Wo2_ref, Ws2_ref, dW_ref,
                  b1o_ref, b1s_ref, b2o_ref, b2s_ref, db_ref,
                  a1_ref, a2r_ref,
                  out_ref, ros_ref, rosa_ref, x2_ref):
    r = lax.broadcasted_iota(jnp.int32, (N, N), 0)
    c = lax.broadcasted_iota(jnp.int32, (N, N), 1)
    eye = (r == c).astype(jnp.float32)

    def make_A(adj_ref):
        adj = adj_ref[:].reshape(NP, NP)[0:N, 0:N]
        deg = jnp.sum(adj, axis=1, keepdims=True) + 1.0
        dinv = 1.0 / jnp.sqrt(deg)
        dinv_r = jnp.sum(dinv * eye, axis=0, keepdims=True)
        return adj * dinv * dinv_r + eye * (dinv * dinv)

    A_o = make_A(adjo_ref)
    A_s = make_A(adjs_ref)

    Wo2a = Wo2_ref[0:H1, :]
    Wo2b = Wo2_ref[H1:2 * H1, :]
    Ws2a = Ws2_ref[0:H1, :]
    Ws2b = Ws2_ref[H1:2 * H1, :]
    b1o = b1o_ref[:]
    b1s = b1s_ref[:]
    b2o = b2o_ref[:]
    b2s = b2s_ref[:]

    def gnn(xwo, xws):
        x1o = jnp.maximum(_dot(A_o, xwo) + b1o, 0.0)
        x1s = jnp.maximum(_dot(A_s, xws) + b1s, 0.0)
        x2o = _dot(A_o, _dot(x1o, Wo2a) + _dot(x1s, Wo2b)) + b2o
        x2s = _dot(A_s, _dot(x1o, Ws2a) + _dot(x1s, Ws2b)) + b2s
        return jnp.concatenate([x2o, x2s], axis=1)

    x2 = gnn(xwoo_ref[:], xwos_ref[:])
    x2a = gnn(xwao_ref[:], xwas_ref[:])

    h = jax.nn.sigmoid(jnp.sum(x2, axis=0, keepdims=True) / N)
    ha = jax.nn.sigmoid(jnp.sum(x2a, axis=0, keepdims=True) / N)

    dW = dW_ref[:]
    db = db_ref[:]
    v = _dot_t(h, dW)
    va = _dot_t(ha, dW)
    sc1 = _dot_t(x2, v)
    sc2 = _dot_t(x2a, v)
    sc1a = _dot_t(x2a, va)
    sc2a = _dot_t(x2, va)
    ros_ref[:] = jnp.concatenate([sc1, sc2], axis=1) + db
    rosa_ref[:] = jnp.concatenate([sc1a, sc2a], axis=1) + db

    drug_k = _gip(x2[0:DRUG, :], DRUG)
    mic_k = _gip(x2[DRUG:N, :], MIC)
    out1 = _dot(drug_k, a1_ref[:])
    out2t = _dot(a2r_ref[:], mic_k)
    out_ref[:] = (out1 + out2t) * 0.5
    x2_ref[:] = x2


@jax.jit
def _fwd(x_o, x_a, edge_index_o, edge_index_s, W_o1, b_o1, W_s1, b_s1,
         W_o2, b_o2, W_s2, b_s2, disc_W, disc_b, alpha1, alpha2):
    idx = pl.pallas_call(_idx_kernel, out_shape=_IDX_SHAPES)(
        edge_index_o, edge_index_s)
    adj0, adj1 = _adj_sc()(idx)
    xw = pl.pallas_call(_pre_kernel, out_shape=_PRE_SHAPES)(
        x_o, x_a, W_o1, W_s1)
    return pl.pallas_call(_mngnn_kernel, out_shape=_OUT_SHAPES)(
        adj0.reshape(NP * 4, NP // 4), adj1.reshape(NP * 4, NP // 4), *xw,
        W_o2, W_s2, disc_W,
        b_o1.reshape(1, H1), b_s1.reshape(1, H1),
        b_o2.reshape(1, H2), b_s2.reshape(1, H2),
        disc_b.reshape(1, 1), alpha1, alpha2.T)


def kernel(x_o, x_a, edge_index_o, edge_index_s, W_o1, b_o1, W_s1, b_s1,
           W_o2, b_o2, W_s2, b_s2, disc_W, disc_b, alpha1, alpha2):
    out, ros, rosa, x2 = _fwd(
        x_o, x_a, edge_index_o, edge_index_s, W_o1, b_o1, W_s1, b_s1,
        W_o2, b_o2, W_s2, b_s2, disc_W, disc_b, alpha1, alpha2)
    return (out, ros, rosa, x2)


# speedup vs baseline: 1.5277x; 1.0202x over previous
import functools

import jax
import jax.numpy as jnp
from jax import lax
from jax.experimental import pallas as pl
from jax.experimental.pallas import tpu as pltpu
from jax.experimental.pallas import tpu_sc as plsc

N = 489
NP = 512
FEAT = 512
H1 = 256
H2 = 128
E = 31296
DRUG = 271
MIC = N - DRUG
GAMMA = 0.5

NSUB = 16
EPAD = 32768
EPS = EPAD // NSUB
IRW = 16
TBL = NP * NP
TSLICE = TBL // NSUB
ZCH = 2048
SENT = TBL - 1


@functools.cache
def _adj_sc():
    mesh = plsc.VectorSubcoreMesh(core_axis_name="c", subcore_axis_name="s")
    return pl.kernel(
        _adj_sc_body,
        out_type=(jax.ShapeDtypeStruct((TBL,), jnp.float32),
                  jax.ShapeDtypeStruct((TBL,), jnp.float32)),
        mesh=mesh,
        scratch_types=[
            pltpu.VMEM((EPS,), jnp.int32),
            pltpu.VMEM((IRW, 128), jnp.int32),
            pltpu.VMEM((128,), jnp.float32),
            pltpu.VMEM((ZCH,), jnp.float32),
            pltpu.VMEM_SHARED((TBL,), jnp.float32),
            pltpu.SemaphoreType.DMA,
            pltpu.SemaphoreType.DMA,
            pltpu.SemaphoreType.DMA,
        ],
    )


def _adj_sc_body(idx_hbm, out0_hbm, out1_hbm,
                 idx1d, idxb, onesb, zerob, table,
                 sem_z, sem_e, sem_s):
    cid = lax.axis_index("c")
    sid = lax.axis_index("s")

    zeros16 = jnp.zeros((16,), jnp.float32)
    ones16 = jnp.ones((16,), jnp.float32)
    for i in range(ZCH // 16):
        zerob[pl.ds(i * 16, 16)] = zeros16
    for i in range(128 // 16):
        onesb[pl.ds(i * 16, 16)] = ones16

    for j in range(TSLICE // ZCH):
        pltpu.async_copy(zerob, table.at[pl.ds(sid * TSLICE + j * ZCH, ZCH)],
                         sem_z)

    base = cid * EPAD + sid * EPS
    pltpu.async_copy(idx_hbm.at[pl.ds(base, EPS)],
                     idx1d.at[pl.ds(0, EPS)], sem_e)
    pltpu.make_async_copy(idx_hbm.at[pl.ds(0, EPS)],
                          idx1d.at[pl.ds(0, EPS)], sem_e).wait()

    for g in range(EPS // 16):
        idxb[g // 8, pl.ds((g % 8) * 16, 16)] = idx1d[pl.ds(g * 16, 16)]

    for j in range(TSLICE // ZCH):
        pltpu.make_async_copy(zerob, table.at[pl.ds(j * ZCH, ZCH)],
                              sem_z).wait()
    plsc.subcore_barrier()

    for r in range(IRW):
        pltpu.async_copy(onesb, table.at[idxb.at[r]], sem_s, add=True)
    for r in range(IRW):
        pltpu.make_async_copy(onesb, table.at[idxb.at[0]], sem_s).wait()

    plsc.subcore_barrier()

    @pl.when(cid == 0)
    def _():
        pltpu.sync_copy(table.at[pl.ds(sid * TSLICE, TSLICE)],
                        out0_hbm.at[pl.ds(sid * TSLICE, TSLICE)])

    @pl.when(cid == 1)
    def _():
        pltpu.sync_copy(table.at[pl.ds(sid * TSLICE, TSLICE)],
                        out1_hbm.at[pl.ds(sid * TSLICE, TSLICE)])


_PRE_SHAPES = tuple(jax.ShapeDtypeStruct((N, H1), jnp.float32)
                    for _ in range(4))

_OUT_SHAPES = (
    jax.ShapeDtypeStruct((DRUG, MIC), jnp.float32),
    jax.ShapeDtypeStruct((4, N), jnp.float32),
    jax.ShapeDtypeStruct((N, H1), jnp.float32),
)


def _dot(a, b):
    return lax.dot_general(a, b, (((1,), (0,)), ((), ())),
                           preferred_element_type=jnp.float32)


def _dot_t(a, b):
    return lax.dot_general(a, b, (((1,), (1,)), ((), ())),
                           preferred_element_type=jnp.float32)


_IDX_SHAPES = jax.ShapeDtypeStruct((2 * EPAD,), jnp.int32)


def _idx_kernel(eio_ref, eis_ref, io_ref):
    pad = jnp.full((1, EPAD - E), SENT, jnp.int32)
    for half, ref in enumerate((eio_ref, eis_ref)):
        idx = ref[1:2, :] * NP + ref[0:1, :]
        io_ref[pl.ds(half * EPAD, EPAD)] = jnp.concatenate(
            [idx, pad], axis=1).reshape(EPAD)


def _pre_kernel(xo_ref, xa_ref, Wo1_ref, Ws1_ref,
                xwoo_ref, xwos_ref, xwao_ref, xwas_ref):
    xo = xo_ref[:]
    xa = xa_ref[:]
    Wo1 = Wo1_ref[:]
    Ws1 = Ws1_ref[:]
    xwoo_ref[:] = _dot(xo, Wo1)
    xwos_ref[:] = _dot(xo, Ws1)
    xwao_ref[:] = _dot(xa, Wo1)
    xwas_ref[:] = _dot(xa, Ws1)


def _gip(y, m):
    mn = jnp.min(y, axis=1, keepdims=True)
    mx = jnp.max(y, axis=1, keepdims=True)
    yn = (y - mn) / (mx - mn)
    k = _dot_t(yn, yn)
    r = lax.broadcasted_iota(jnp.int32, (m, m), 0)
    c = lax.broadcasted_iota(jnp.int32, (m, m), 1)
    eye = (r == c).astype(jnp.float32)
    dcol = jnp.sum(k * eye, axis=1, keepdims=True)
    md = jnp.sum(dcol) / m
    k = k / md
    dcol = dcol / md
    drow = jnp.sum(k * eye, axis=0, keepdims=True)
    dist = dcol + drow - 2.0 * k
    kk = jnp.abs(jnp.exp(dist * (-GAMMA)))
    mp = jnp.min(jnp.where(kk > 0.0, kk, jnp.inf))
    min_v = jnp.where(mp == jnp.inf, 0.0, mp)
    kk = jnp.where(kk == 0.0, min_v, kk)
    dg_c = jnp.sqrt(jnp.sum(kk * eye, axis=1, keepdims=True))
    dg_r = jnp.sqrt(jnp.sum(kk * eye, axis=0, keepdims=True))
    return kk / (dg_c * dg_r)


def _mngnn_kernel(adjo_ref, adjs_ref,
                  xwoo_ref, xwos_ref, xwao_ref, xwas_ref,
                  Wo2_ref, Ws2_ref, dW_ref,
                  b1o_ref, b1s_ref, b2o_ref, b2s_ref, db_ref,
                  a1_ref, a2r_ref,
                  out_ref, ret_ref, x2_ref):
    r = lax.broadcasted_iota(jnp.int32, (N, N), 0)
    c = lax.broadcasted_iota(jnp.int32, (N, N), 1)
    eye = (r == c).astype(jnp.float32)

    def make_A(adj_ref):
        adj = adj_ref[:].reshape(NP, NP)[0:N, 0:N]
        deg = jnp.sum(adj, axis=1, keepdims=True) + 1.0
        dinv = 1.0 / jnp.sqrt(deg)
        dinv_r = jnp.sum(dinv * eye, axis=0, keepdims=True)
        return adj * dinv * dinv_r + eye * (dinv * dinv)

    A_o = make_A(adjo_ref)
    A_s = make_A(adjs_ref)

    Wo2a = Wo2_ref[0:H1, :]
    Wo2b = Wo2_ref[H1:2 * H1, :]
    Ws2a = Ws2_ref[0:H1, :]
    Ws2b = Ws2_ref[H1:2 * H1, :]
    b1o = b1o_ref[:]
    b1s = b1s_ref[:]
    b2o = b2o_ref[:]
    b2s = b2s_ref[:]

    def gnn(xwo, xws):
        x1o = jnp.maximum(_dot(A_o, xwo) + b1o, 0.0)
        x1s = jnp.maximum(_dot(A_s, xws) + b1s, 0.0)
        x2o = _dot(A_o, _dot(x1o, Wo2a) + _dot(x1s, Wo2b)) + b2o
        x2s = _dot(A_s, _dot(x1o, Ws2a) + _dot(x1s, Ws2b)) + b2s
        return jnp.concatenate([x2o, x2s], axis=1)

    x2 = gnn(xwoo_ref[:], xwos_ref[:])
    x2a = gnn(xwao_ref[:], xwas_ref[:])

    h = jax.nn.sigmoid(jnp.sum(x2, axis=0, keepdims=True) / N)
    ha = jax.nn.sigmoid(jnp.sum(x2a, axis=0, keepdims=True) / N)

    dW = dW_ref[:]
    db = db_ref[:]
    v = _dot_t(h, dW)
    va = _dot_t(ha, dW)
    sc1 = _dot_t(v, x2)
    sc2 = _dot_t(v, x2a)
    sc1a = _dot_t(va, x2a)
    sc2a = _dot_t(va, x2)
    ret_ref[:] = jnp.concatenate([sc1, sc2, sc1a, sc2a], axis=0) + db

    drug_k = _gip(x2[0:DRUG, :], DRUG)
    mic_k = _gip(x2[DRUG:N, :], MIC)
    out1 = _dot(drug_k, a1_ref[:])
    out2t = _dot(a2r_ref[:], mic_k)
    out_ref[:] = (out1 + out2t) * 0.5
    x2_ref[:] = x2


@jax.jit
def _fwd(x_o, x_a, edge_index_o, edge_index_s, W_o1, b_o1, W_s1, b_s1,
         W_o2, b_o2, W_s2, b_s2, disc_W, disc_b, alpha1, alpha2):
    idx = pl.pallas_call(_idx_kernel, out_shape=_IDX_SHAPES)(
        edge_index_o, edge_index_s)
    adj0, adj1 = _adj_sc()(idx)
    xw = pl.pallas_call(_pre_kernel, out_shape=_PRE_SHAPES)(
        x_o, x_a, W_o1, W_s1)
    out, ret4, x2 = pl.pallas_call(_mngnn_kernel, out_shape=_OUT_SHAPES)(
        adj0.reshape(NP * 4, NP // 4), adj1.reshape(NP * 4, NP // 4), *xw,
        W_o2, W_s2, disc_W,
        b_o1.reshape(1, H1), b_s1.reshape(1, H1),
        b_o2.reshape(1, H2), b_s2.reshape(1, H2),
        disc_b.reshape(1, 1), alpha1, alpha2.T)
    return out, ret4[0:2].T, ret4[2:4].T, x2


def kernel(x_o, x_a, edge_index_o, edge_index_s, W_o1, b_o1, W_s1, b_s1,
           W_o2, b_o2, W_s2, b_s2, disc_W, disc_b, alpha1, alpha2):
    out, ros, rosa, x2 = _fwd(
        x_o, x_a, edge_index_o, edge_index_s, W_o1, b_o1, W_s1, b_s1,
        W_o2, b_o2, W_s2, b_s2, disc_W, disc_b, alpha1, alpha2)
    return (out, ros, rosa, x2)
